# baseline (speedup 1.0000x reference)
import functools

import jax
import jax.numpy as jnp
from jax import lax
from jax.experimental import pallas as pl
from jax.experimental.pallas import tpu as pltpu
from jax.experimental.pallas import tpu_sc as plsc

N = 10000
E = 160000
B = 64
EMB = 256
HALF = 128
NL = 3


BN = 1000
BE = 2000


def _embed_body(x_ref, w_ref, b_ref, h_ref):
    z = jnp.dot(x_ref[...], w_ref[...], preferred_element_type=jnp.float32)
    h_ref[...] = jnp.maximum(z + b_ref[...], 0.0)


def _embed(x, w_in, b_in):
    return pl.pallas_call(
        _embed_body,
        grid=(N // BN,),
        in_specs=[
            pl.BlockSpec((BN, 128), lambda i: (i, 0)),
            pl.BlockSpec((128, EMB), lambda i: (0, 0)),
            pl.BlockSpec((1, EMB), lambda i: (0, 0)),
        ],
        out_specs=pl.BlockSpec((BN, EMB), lambda i: (i, 0)),
        out_shape=jax.ShapeDtypeStruct((N, EMB), jnp.float32),
    )(x, w_in, b_in)


def _mcalc_body(e_ref, w_ref, m0_ref, m1_ref):
    z = jnp.dot(e_ref[...], w_ref[...], preferred_element_type=jnp.float32)
    z = jnp.maximum(z, 0.0)
    m0_ref[...] = z[:, :HALF]
    m1_ref[...] = z[:, HALF:]


def _mcalc(e, w_e_l):
    return pl.pallas_call(
        _mcalc_body,
        grid=(E // BE,),
        in_specs=[
            pl.BlockSpec((BE, 16), lambda i: (i, 0)),
            pl.BlockSpec((16, EMB), lambda i: (0, 0)),
        ],
        out_specs=[
            pl.BlockSpec((BE, HALF), lambda i: (i, 0)),
            pl.BlockSpec((BE, HALF), lambda i: (i, 0)),
        ],
        out_shape=[
            jax.ShapeDtypeStruct((E, HALF), jnp.float32),
            jax.ShapeDtypeStruct((E, HALF), jnp.float32),
        ],
    )(e, w_e_l)


def _dense_body(h_ref, a0_ref, a1_ref, w_ref, b_ref, o_ref):
    a = jnp.concatenate([a0_ref[...], a1_ref[...]], axis=1)
    z = jnp.dot(h_ref[...] + a, w_ref[...], preferred_element_type=jnp.float32)
    o_ref[...] = jnp.maximum(z + b_ref[...], 0.0)


def _dense(h, a0, a1, w_h_l, b_h_l):
    return pl.pallas_call(
        _dense_body,
        grid=(N // BN,),
        in_specs=[
            pl.BlockSpec((BN, EMB), lambda i: (i, 0)),
            pl.BlockSpec((BN, HALF), lambda i: (i, 0)),
            pl.BlockSpec((BN, HALF), lambda i: (i, 0)),
            pl.BlockSpec((EMB, EMB), lambda i: (0, 0)),
            pl.BlockSpec((1, EMB), lambda i: (0, 0)),
        ],
        out_specs=pl.BlockSpec((BN, EMB), lambda i: (i, 0)),
        out_shape=jax.ShapeDtypeStruct((N, EMB), jnp.float32),
    )(h, a0, a1, w_h_l, b_h_l)


def _readout_body(seg_ref, h_ref, m_ref, cnt_ref):
    i = pl.program_id(0)
    ni = pl.num_programs(0)

    @pl.when(i == 0)
    def _():
        m_ref[...] = jnp.zeros_like(m_ref)
        cnt_ref[...] = jnp.zeros_like(cnt_ref)

    seg = seg_ref[0]
    oh = (lax.broadcasted_iota(jnp.int32, (B, BN), 0) == seg).astype(jnp.float32)
    m_ref[...] += jnp.dot(oh, h_ref[...], preferred_element_type=jnp.float32)
    cnt_ref[...] += jnp.broadcast_to(
        jnp.sum(oh, axis=1, keepdims=True), (B, EMB))

    @pl.when(i == ni - 1)
    def _():
        m_ref[...] = m_ref[...] / jnp.maximum(cnt_ref[...], 1.0)


def _readout(seg3d, h):
    return pl.pallas_call(
        _readout_body,
        grid=(N // BN,),
        in_specs=[
            pl.BlockSpec((1, 1, BN), lambda i: (i, 0, 0)),
            pl.BlockSpec((BN, EMB), lambda i: (i, 0)),
        ],
        out_specs=pl.BlockSpec((B, EMB), lambda i: (0, 0)),
        out_shape=jax.ShapeDtypeStruct((B, EMB), jnp.float32),
        scratch_shapes=[pltpu.VMEM((B, EMB), jnp.float32)],
    )(seg3d, h)


def _att_body(a0_ref, a1_ref, p_ref, wq_ref, wk_ref, wo_ref, bo_ref,
              out_ref, attr_ref, attp_ref, rvec_ref):
    a0 = a0_ref[...]
    a1 = a1_ref[...]
    p = p_ref[...]
    a2 = a0 + a1
    wq = wq_ref[...]
    wk = wk_ref[...]

    scale = 0.125
    qp = jnp.dot(p, wq, preferred_element_type=jnp.float32)
    s0 = jnp.sum(qp * jnp.dot(a0, wk, preferred_element_type=jnp.float32),
                 axis=1, keepdims=True) * scale
    s1 = jnp.sum(qp * jnp.dot(a1, wk, preferred_element_type=jnp.float32),
                 axis=1, keepdims=True) * scale
    s2 = jnp.sum(qp * jnp.dot(a2, wk, preferred_element_type=jnp.float32),
                 axis=1, keepdims=True) * scale
    mx = jnp.maximum(jnp.maximum(s0, s1), s2)
    e0 = jnp.exp(s0 - mx)
    e1 = jnp.exp(s1 - mx)
    e2 = jnp.exp(s2 - mx)
    den = e0 + e1 + e2
    w0 = e0 / den
    w1 = e1 / den
    w2 = e2 / den
    attr_ref[...] = jnp.concatenate([w0, w1, w2], axis=1)
    attp_ref[...] = jnp.ones_like(attp_ref)
    rvec_ref[...] = (w0 * a0 + w1 * a1 + w2 * a2) - p

    wo = wo_ref[...]
    bo = bo_ref[...]
    out_ref[0] = jnp.dot(a0 - p, wo, preferred_element_type=jnp.float32) + bo
    out_ref[1] = jnp.dot(a1 - p, wo, preferred_element_type=jnp.float32) + bo


def _att(means, wq, wk, w_out, b_out):
    return pl.pallas_call(
        _att_body,
        out_shape=[
            jax.ShapeDtypeStruct((2, B, 1), jnp.float32),
            jax.ShapeDtypeStruct((B, 3), jnp.float32),
            jax.ShapeDtypeStruct((B, 1), jnp.float32),
            jax.ShapeDtypeStruct((B, EMB), jnp.float32),
        ],
    )(means[0], means[1], means[2], wq, wk, w_out, b_out.reshape(1, 1))


NTILES = 16
KE = 128
EPT = E // NTILES
NV = EPT // 16
SCAP = EPT + 272
EP_TOT = E + 256 * 127
TRASH = N
NACC = N + NTILES
RPT = 624
REM_BASE = RPT * NTILES
REM = NACC - REM_BASE

_IOTA16 = None


def _lane(vec, idx):
    m = lax.broadcasted_iota(jnp.int32, (16,), 0) == idx
    return jnp.max(jnp.where(m, vec, 0))


def _bucket_body(src, dst, srcP, dstP, eidP, grid_h,
                 sbuf, dbuf, ownb, srcS, dstS, eidS, cntv, gvm, gridspm, sem):
    c = lax.axis_index("c")
    s = lax.axis_index("s")

    @pl.when(c == 0)
    def _():
        iota16 = lax.broadcasted_iota(jnp.int32, (16,), 0)
        base_e = s * EPT
        pltpu.sync_copy(src.at[pl.ds(base_e, EPT)], sbuf)
        pltpu.sync_copy(dst.at[pl.ds(base_e, EPT)], dbuf)

        def _own(v, carry):
            dv = dbuf[pl.ds(v * 16, 16)]
            ownb[pl.ds(v * 16, 16)] = jnp.minimum(
                jnp.right_shift(dv * 26887, 24), 15)
            return carry

        lax.fori_loop(0, NV, _own, 0)
        if True:
            return

        cnt_vec = jnp.zeros((16,), jnp.int32)
        for o in range(NTILES):
            def _compact(v, off):
                m = ownb[pl.ds(v * 16, 16)] == o
                sv = sbuf[pl.ds(v * 16, 16)]
                dv = dbuf[pl.ds(v * 16, 16)]
                ev = base_e + v * 16 + iota16
                mi = m.astype(jnp.int32)
                pos = off + plsc.cumsum(mi) - 1
                plsc.store_scatter(srcS, [pos], sv, mask=m)
                plsc.store_scatter(dstS, [pos], dv, mask=m)
                plsc.store_scatter(eidS, [pos], ev, mask=m)
                return off + jnp.sum(mi)

            off = lax.fori_loop(0, NV, _compact, 0)
            for k in range(8):
                pos = off + k * 16 + iota16
                plsc.store_scatter(srcS, [pos], jnp.zeros((16,), jnp.int32))
                plsc.store_scatter(dstS, [pos],
                                   jnp.full((16,), TRASH + o, jnp.int32))
                plsc.store_scatter(eidS, [pos], jnp.zeros((16,), jnp.int32))
            cnt_vec = jnp.where(iota16 == o, off, cnt_vec)

        cntv[...] = cnt_vec
        pltpu.sync_copy(cntv, gridspm.at[s])
        plsc.subcore_barrier()
        pltpu.sync_copy(gridspm, gvm)

        _BISECT = True
        if _BISECT:
            return
        run_off = jnp.zeros((16,), jnp.int32)
        tot = jnp.zeros((16,), jnp.int32)
        for t in range(NTILES):
            row = gvm[t]
            rp = (row + 127) & (-128)
            run_off = jnp.where(s == t, tot, run_off)
            tot = tot + rp
        reg_start = plsc.cumsum(tot) - tot
        my_dest = reg_start + run_off

        for o in range(NTILES):
            dest0 = _lane(my_dest, o)
            cnt_o = _lane(cnt_vec, o)
            nfull = jnp.right_shift(cnt_o + 127, 7)

            def _cp(i, carry):
                so = pl.multiple_of(i * 128, 128)
                d = pl.multiple_of(dest0 + i * 128, 128)
                pltpu.sync_copy(srcS.at[pl.ds(so, 128)],
                                srcP.at[pl.ds(d, 128)])
                pltpu.sync_copy(dstS.at[pl.ds(so, 128)],
                                dstP.at[pl.ds(d, 128)])
                pltpu.sync_copy(eidS.at[pl.ds(so, 128)],
                                eidP.at[pl.ds(d, 128)])
                return carry

            lax.fori_loop(0, nfull, _cp, 0)

        @pl.when(s == 0)
        def _():
            pltpu.sync_copy(gridspm, grid_h)


_bucket = functools.partial(
    pl.kernel,
    out_type=[
        jax.ShapeDtypeStruct((EP_TOT,), jnp.int32),
        jax.ShapeDtypeStruct((EP_TOT,), jnp.int32),
        jax.ShapeDtypeStruct((EP_TOT,), jnp.int32),
        jax.ShapeDtypeStruct((NTILES, NTILES), jnp.int32),
    ],
    mesh=plsc.VectorSubcoreMesh(core_axis_name="c", subcore_axis_name="s"),
    scratch_types=[
        pltpu.VMEM((EPT,), jnp.int32),
        pltpu.VMEM((EPT,), jnp.int32),
        pltpu.VMEM((EPT,), jnp.int32),
        pltpu.VMEM((SCAP,), jnp.int32),
        pltpu.VMEM((SCAP,), jnp.int32),
        pltpu.VMEM((SCAP,), jnp.int32),
        pltpu.VMEM((16,), jnp.int32),
        pltpu.VMEM((NTILES, NTILES), jnp.int32),
        pltpu.VMEM_SHARED((NTILES, NTILES), jnp.int32),
        pltpu.SemaphoreType.DMA,
    ],
)(_bucket_body)


def _spmm_body(h0, h1, m0, m1, srcP, dstP, eidP, grid_h, out0, out1,
               srcv, dstv, eidv, buf, mbuf, gvm, acc, semg, semm):
    c = lax.axis_index("c")
    s = lax.axis_index("s")

    def _zrow(r, carry):
        for q in range(8):
            buf[r, pl.ds(q * 16, 16)] = jnp.zeros((16,), jnp.float32)
        return carry

    lax.fori_loop(0, KE, _zrow, 0)
    rbase = s * RPT
    for j in range(4):
        pltpu.sync_copy(buf.at[pl.ds(0, 128)],
                        acc.at[pl.ds(rbase + j * 128, 128)])
    pltpu.sync_copy(buf.at[pl.ds(0, 112)], acc.at[pl.ds(rbase + 512, 112)])

    @pl.when(s == 0)
    def _():
        pltpu.sync_copy(buf.at[pl.ds(0, REM)], acc.at[pl.ds(REM_BASE, REM)])

    pltpu.sync_copy(grid_h, gvm)
    run = jnp.zeros((16,), jnp.int32)
    tot = jnp.zeros((16,), jnp.int32)
    for t in range(NTILES):
        rp = (gvm[t] + 127) & (-128)
        tot = tot + rp
    reg_start = plsc.cumsum(tot) - tot
    my_start = _lane(reg_start, s)
    nch = jnp.right_shift(_lane(tot, s), 7)

    plsc.subcore_barrier()

    def _half(h, m, out):
        def _chunk(i, carry):
            eb = pl.multiple_of(my_start + i * KE, KE)
            pltpu.sync_copy(srcP.at[pl.ds(eb, KE)], srcv)
            pltpu.sync_copy(dstP.at[pl.ds(eb, KE)], dstv)
            pltpu.sync_copy(eidP.at[pl.ds(eb, KE)], eidv)
            cm = pltpu.async_copy(m.at[eidv], mbuf, semm)
            cg = pltpu.async_copy(h.at[srcv], buf, semg)
            cm.wait()
            cg.wait()

            def _add(r, carry):
                for q in range(8):
                    plsc.addupdate(buf.at[r, pl.ds(q * 16, 16)],
                                   mbuf[r, pl.ds(q * 16, 16)])
                return carry

            lax.fori_loop(0, KE, _add, 0)
            pltpu.sync_copy(buf, acc.at[dstv], add=True)
            return carry

        lax.fori_loop(0, nch, _chunk, 0)
        plsc.subcore_barrier()
        pltpu.sync_copy(acc.at[pl.ds(rbase, RPT)], out.at[pl.ds(rbase, RPT)])

        @pl.when(s == 0)
        def _():
            pltpu.sync_copy(acc.at[pl.ds(REM_BASE, REM - NTILES)],
                            out.at[pl.ds(REM_BASE, REM - NTILES)])

    @pl.when(c == 0)
    def _():
        _half(h0, m0, out0)

    @pl.when(c == 1)
    def _():
        _half(h1, m1, out1)


_spmm = functools.partial(
    pl.kernel,
    out_type=[
        jax.ShapeDtypeStruct((N, HALF), jnp.float32),
        jax.ShapeDtypeStruct((N, HALF), jnp.float32),
    ],
    mesh=plsc.VectorSubcoreMesh(core_axis_name="c", subcore_axis_name="s"),
    scratch_types=[
        pltpu.VMEM((KE,), jnp.int32),
        pltpu.VMEM((KE,), jnp.int32),
        pltpu.VMEM((KE,), jnp.int32),
        pltpu.VMEM((KE, HALF), jnp.float32),
        pltpu.VMEM((KE, HALF), jnp.float32),
        pltpu.VMEM((NTILES, NTILES), jnp.int32),
        pltpu.VMEM_SHARED((NACC, HALF), jnp.float32),
        pltpu.SemaphoreType.DMA,
        pltpu.SemaphoreType.DMA,
    ],
)(_spmm_body)


def kernel(x_r, e_r, x_p, e_p, edge_index_r, edge_index_p, seg_r, seg_p,
           r_dummy, p_dummy, W_in, b_in, W_e, W_h, b_h, Wq, Wk, W_out, b_out):
    xs = [x_r[0], x_r[1], x_p[0]]
    es = [e_r[0], e_r[1], e_p[0]]
    srcs = [edge_index_r[0, 0].astype(jnp.int32),
            edge_index_r[1, 0].astype(jnp.int32),
            edge_index_p[0, 0].astype(jnp.int32)]
    dsts = [edge_index_r[0, 1].astype(jnp.int32),
            edge_index_r[1, 1].astype(jnp.int32),
            edge_index_p[0, 1].astype(jnp.int32)]
    segs = [seg_r[0].astype(jnp.int32).reshape(N // BN, 1, BN),
            seg_r[1].astype(jnp.int32).reshape(N // BN, 1, BN),
            seg_p[0].astype(jnp.int32).reshape(N // BN, 1, BN)]

    b_in2 = b_in.reshape(1, EMB)

    buckets = [None for g in range(3)]
    hs = [_embed(x, W_in, b_in2) for x in xs]
    ms = [[_mcalc(es[g], W_e[l]) for l in range(NL)] for g in range(3)]

    for l in range(NL):
        w_l = W_h[l]
        b_l = b_h[l].reshape(1, EMB)
        for g in range(3):
            if True:
                msg = hs[g][srcs[g]] + jnp.concatenate(
                    [ms[g][l][0], ms[g][l][1]], 1)
                agg = jnp.zeros((N, EMB), jnp.float32).at[dsts[g]].add(msg)
                a0, a1 = agg[:, :HALF], agg[:, HALF:]
            else:
                srcP, dstP, eidP, grid = buckets[g]
                h0 = hs[g][:, :HALF]
                h1 = hs[g][:, HALF:]
                a0, a1 = _spmm(h0, h1, ms[g][l][0], ms[g][l][1],
                               srcP, dstP, eidP, grid)
            hs[g] = _dense(hs[g], a0, a1, w_l, b_l)

    means = [_readout(segs[g], hs[g]) for g in range(3)]
    return _att(means, Wq, Wk, W_out, b_out)


# speedup vs baseline: 1.1726x; 1.1726x over previous
import functools

import jax
import jax.numpy as jnp
from jax import lax
from jax.experimental import pallas as pl
from jax.experimental.pallas import tpu as pltpu
from jax.experimental.pallas import tpu_sc as plsc

N = 10000
E = 160000
B = 64
EMB = 256
HALF = 128
NL = 3


BN = 1000
BE = 2000


def _embed_body(x_ref, w_ref, b_ref, h_ref):
    z = jnp.dot(x_ref[...], w_ref[...], preferred_element_type=jnp.float32)
    h_ref[...] = jnp.maximum(z + b_ref[...], 0.0)


def _embed(x, w_in, b_in):
    return pl.pallas_call(
        _embed_body,
        grid=(N // BN,),
        in_specs=[
            pl.BlockSpec((BN, 128), lambda i: (i, 0)),
            pl.BlockSpec((128, EMB), lambda i: (0, 0)),
            pl.BlockSpec((1, EMB), lambda i: (0, 0)),
        ],
        out_specs=pl.BlockSpec((BN, EMB), lambda i: (i, 0)),
        out_shape=jax.ShapeDtypeStruct((N, EMB), jnp.float32),
    )(x, w_in, b_in)


def _mcalc_body(e_ref, w_ref, m0_ref, m1_ref):
    z = jnp.dot(e_ref[...], w_ref[...], preferred_element_type=jnp.float32)
    z = jnp.maximum(z, 0.0)
    m0_ref[...] = z[:, :HALF]
    m1_ref[...] = z[:, HALF:]


def _mcalc(e, w_e_l):
    return pl.pallas_call(
        _mcalc_body,
        grid=(E // BE,),
        in_specs=[
            pl.BlockSpec((BE, 16), lambda i: (i, 0)),
            pl.BlockSpec((16, EMB), lambda i: (0, 0)),
        ],
        out_specs=[
            pl.BlockSpec((BE, HALF), lambda i: (i, 0)),
            pl.BlockSpec((BE, HALF), lambda i: (i, 0)),
        ],
        out_shape=[
            jax.ShapeDtypeStruct((E, HALF), jnp.float32),
            jax.ShapeDtypeStruct((E, HALF), jnp.float32),
        ],
    )(e, w_e_l)


def _dense_body(h_ref, a0_ref, a1_ref, w_ref, b_ref, o_ref):
    a = jnp.concatenate([a0_ref[...], a1_ref[...]], axis=1)
    z = jnp.dot(h_ref[...] + a, w_ref[...], preferred_element_type=jnp.float32)
    o_ref[...] = jnp.maximum(z + b_ref[...], 0.0)


def _dense(h, a0, a1, w_h_l, b_h_l):
    return pl.pallas_call(
        _dense_body,
        grid=(N // BN,),
        in_specs=[
            pl.BlockSpec((BN, EMB), lambda i: (i, 0)),
            pl.BlockSpec((BN, HALF), lambda i: (i, 0)),
            pl.BlockSpec((BN, HALF), lambda i: (i, 0)),
            pl.BlockSpec((EMB, EMB), lambda i: (0, 0)),
            pl.BlockSpec((1, EMB), lambda i: (0, 0)),
        ],
        out_specs=pl.BlockSpec((BN, EMB), lambda i: (i, 0)),
        out_shape=jax.ShapeDtypeStruct((N, EMB), jnp.float32),
    )(h, a0, a1, w_h_l, b_h_l)


def _readout_body(seg_ref, h_ref, m_ref, cnt_ref):
    i = pl.program_id(0)
    ni = pl.num_programs(0)

    @pl.when(i == 0)
    def _():
        m_ref[...] = jnp.zeros_like(m_ref)
        cnt_ref[...] = jnp.zeros_like(cnt_ref)

    seg = seg_ref[0]
    oh = (lax.broadcasted_iota(jnp.int32, (B, BN), 0) == seg).astype(jnp.float32)
    m_ref[...] += jnp.dot(oh, h_ref[...], preferred_element_type=jnp.float32)
    cnt_ref[...] += jnp.broadcast_to(
        jnp.sum(oh, axis=1, keepdims=True), (B, EMB))

    @pl.when(i == ni - 1)
    def _():
        m_ref[...] = m_ref[...] / jnp.maximum(cnt_ref[...], 1.0)


def _readout(seg3d, h):
    return pl.pallas_call(
        _readout_body,
        grid=(N // BN,),
        in_specs=[
            pl.BlockSpec((1, 1, BN), lambda i: (i, 0, 0)),
            pl.BlockSpec((BN, EMB), lambda i: (i, 0)),
        ],
        out_specs=pl.BlockSpec((B, EMB), lambda i: (0, 0)),
        out_shape=jax.ShapeDtypeStruct((B, EMB), jnp.float32),
        scratch_shapes=[pltpu.VMEM((B, EMB), jnp.float32)],
    )(seg3d, h)


def _att_body(a0_ref, a1_ref, p_ref, wq_ref, wk_ref, wo_ref, bo_ref,
              out_ref, attr_ref, attp_ref, rvec_ref):
    a0 = a0_ref[...]
    a1 = a1_ref[...]
    p = p_ref[...]
    a2 = a0 + a1
    wq = wq_ref[...]
    wk = wk_ref[...]

    scale = 0.125
    qp = jnp.dot(p, wq, preferred_element_type=jnp.float32)
    s0 = jnp.sum(qp * jnp.dot(a0, wk, preferred_element_type=jnp.float32),
                 axis=1, keepdims=True) * scale
    s1 = jnp.sum(qp * jnp.dot(a1, wk, preferred_element_type=jnp.float32),
                 axis=1, keepdims=True) * scale
    s2 = jnp.sum(qp * jnp.dot(a2, wk, preferred_element_type=jnp.float32),
                 axis=1, keepdims=True) * scale
    mx = jnp.maximum(jnp.maximum(s0, s1), s2)
    e0 = jnp.exp(s0 - mx)
    e1 = jnp.exp(s1 - mx)
    e2 = jnp.exp(s2 - mx)
    den = e0 + e1 + e2
    w0 = e0 / den
    w1 = e1 / den
    w2 = e2 / den
    attr_ref[...] = jnp.concatenate([w0, w1, w2], axis=1)
    attp_ref[...] = jnp.ones_like(attp_ref)
    rvec_ref[...] = (w0 * a0 + w1 * a1 + w2 * a2) - p

    wo = wo_ref[...]
    bo = bo_ref[...]
    out_ref[0] = jnp.dot(a0 - p, wo, preferred_element_type=jnp.float32) + bo
    out_ref[1] = jnp.dot(a1 - p, wo, preferred_element_type=jnp.float32) + bo


def _att(means, wq, wk, w_out, b_out):
    return pl.pallas_call(
        _att_body,
        out_shape=[
            jax.ShapeDtypeStruct((2, B, 1), jnp.float32),
            jax.ShapeDtypeStruct((B, 3), jnp.float32),
            jax.ShapeDtypeStruct((B, 1), jnp.float32),
            jax.ShapeDtypeStruct((B, EMB), jnp.float32),
        ],
    )(means[0], means[1], means[2], wq, wk, w_out, b_out.reshape(1, 1))


NTILES = 16
KE = 128
EPT = E // NTILES
EP_TOT = 163840
EPW = EP_TOT // NTILES
TRASH = N
NACC = N + NTILES
RPT = 624
REM_BASE = RPT * NTILES
REM = NACC - REM_BASE
REMW = N - REM_BASE


def _lane(vec, idx):
    m = lax.broadcasted_iota(jnp.int32, (16,), 0) == idx
    return jnp.max(jnp.where(m, vec, 0))


def _pos_body(dst_ref, pos_ref, st_ref, nch_ref, cntc, runc):
    p = pl.program_id(0)
    i = pl.program_id(1)
    ni = pl.num_programs(1)

    @pl.when(jnp.logical_and(p == 0, i == 0))
    def _():
        cntc[...] = jnp.zeros_like(cntc)
        runc[...] = jnp.zeros_like(runc)

    d = dst_ref[0]
    own = jnp.minimum(jnp.right_shift(d * 26887, 24), 15)
    oh = (lax.broadcasted_iota(jnp.int32, (16, BE), 0) == own
          ).astype(jnp.float32)

    @pl.when(p == 0)
    def _():
        cntc[...] += jnp.broadcast_to(
            jnp.sum(oh, axis=1, keepdims=True), (16, 128))

    @pl.when(p == 1)
    def _():
        cp = cntc[:, :1]
        cp128 = jnp.floor((cp + 127.0) * (1.0 / 128.0)) * 128.0
        tl = (lax.broadcasted_iota(jnp.int32, (16, 16), 0)
              > lax.broadcasted_iota(jnp.int32, (16, 16), 1)
              ).astype(jnp.float32)
        starts = jnp.dot(tl, cp128, preferred_element_type=jnp.float32)
        mm = (lax.broadcasted_iota(jnp.int32, (BE, BE), 0)
              <= lax.broadcasted_iota(jnp.int32, (BE, BE), 1)
              ).astype(jnp.float32)
        pref = jnp.dot(oh, mm, preferred_element_type=jnp.float32)
        base = starts + runc[:, :1] - 1.0
        posf = jnp.sum(oh * (pref + base), axis=0, keepdims=True)
        pos_ref[0] = posf.astype(jnp.int32)
        runc[...] += jnp.broadcast_to(
            jnp.sum(oh, axis=1, keepdims=True), (16, 128))

        @pl.when(i == ni - 1)
        def _():
            st_ref[...] = starts.astype(jnp.int32)
            nch_ref[...] = (cp128 * (1.0 / 128.0)).astype(jnp.int32)


def _pos(dst3d):
    return pl.pallas_call(
        _pos_body,
        grid=(2, E // BE),
        in_specs=[pl.BlockSpec((1, 1, BE), lambda p, i: (i, 0, 0))],
        out_specs=[
            pl.BlockSpec((1, 1, BE), lambda p, i: (i, 0, 0)),
            pl.BlockSpec((16, 1), lambda p, i: (0, 0)),
            pl.BlockSpec((16, 1), lambda p, i: (0, 0)),
        ],
        out_shape=[
            jax.ShapeDtypeStruct((E // BE, 1, BE), jnp.int32),
            jax.ShapeDtypeStruct((16, 1), jnp.int32),
            jax.ShapeDtypeStruct((16, 1), jnp.int32),
        ],
        scratch_shapes=[pltpu.VMEM((16, 128), jnp.float32),
                        pltpu.VMEM((16, 128), jnp.float32)],
    )(dst3d)


def _scb_body(src, dst, pos, srcP, dstP, eidP,
              sv, dv, pv, ev, sv2, dv2, pv2, ev2, fbuf, sS, dS, eS, sem):
    c = lax.axis_index("c")
    s = lax.axis_index("s")

    @pl.when(c == 0)
    def _():
        iota16 = lax.broadcasted_iota(jnp.int32, (16,), 0)

        def _fill(r, carry):
            fbuf[pl.ds(r * 16, 16)] = jnp.full((16,), TRASH, jnp.int32)
            return carry

        lax.fori_loop(0, 64, _fill, 0)
        wb = s * EPW
        for j in range(10):
            pltpu.sync_copy(fbuf, dS.at[pl.ds(wb + j * 1024, 1024)])

        def _zfill(r, carry):
            fbuf[pl.ds(r * 16, 16)] = jnp.zeros((16,), jnp.int32)
            return carry

        lax.fori_loop(0, 64, _zfill, 0)
        for j in range(10):
            pltpu.sync_copy(fbuf, sS.at[pl.ds(wb + j * 1024, 1024)])
            pltpu.sync_copy(fbuf, eS.at[pl.ds(wb + j * 1024, 1024)])
        plsc.subcore_barrier()

        base_e = s * EPT
        nfull = EPT // KE

        def _chunk(i, carry):
            eb = pl.multiple_of(base_e + i * KE, 16)
            pltpu.sync_copy(src.at[pl.ds(eb, KE)], sv)
            pltpu.sync_copy(dst.at[pl.ds(eb, KE)], dv)
            pltpu.sync_copy(pos.at[pl.ds(eb, KE)], pv)
            for q in range(8):
                ev[pl.ds(q * 16, 16)] = eb + q * 16 + iota16
            pltpu.sync_copy(sv, sS.at[pv])
            pltpu.sync_copy(dv, dS.at[pv])
            pltpu.sync_copy(ev, eS.at[pv])
            return carry

        lax.fori_loop(0, nfull, _chunk, 0)
        ebr = base_e + nfull * KE
        pltpu.sync_copy(src.at[pl.ds(ebr, 16)], sv2)
        pltpu.sync_copy(dst.at[pl.ds(ebr, 16)], dv2)
        pltpu.sync_copy(pos.at[pl.ds(ebr, 16)], pv2)
        ev2[...] = ebr + iota16
        pltpu.sync_copy(sv2, sS.at[pv2])
        pltpu.sync_copy(dv2, dS.at[pv2])
        pltpu.sync_copy(ev2, eS.at[pv2])
        plsc.subcore_barrier()

        pltpu.sync_copy(sS.at[pl.ds(wb, EPW)], srcP.at[pl.ds(wb, EPW)])
        pltpu.sync_copy(dS.at[pl.ds(wb, EPW)], dstP.at[pl.ds(wb, EPW)])
        pltpu.sync_copy(eS.at[pl.ds(wb, EPW)], eidP.at[pl.ds(wb, EPW)])


_scb = functools.partial(
    pl.kernel,
    out_type=[
        jax.ShapeDtypeStruct((EP_TOT,), jnp.int32),
        jax.ShapeDtypeStruct((EP_TOT,), jnp.int32),
        jax.ShapeDtypeStruct((EP_TOT,), jnp.int32),
    ],
    mesh=plsc.VectorSubcoreMesh(core_axis_name="c", subcore_axis_name="s"),
    scratch_types=[
        pltpu.VMEM((KE,), jnp.int32),
        pltpu.VMEM((KE,), jnp.int32),
        pltpu.VMEM((KE,), jnp.int32),
        pltpu.VMEM((KE,), jnp.int32),
        pltpu.VMEM((16,), jnp.int32),
        pltpu.VMEM((16,), jnp.int32),
        pltpu.VMEM((16,), jnp.int32),
        pltpu.VMEM((16,), jnp.int32),
        pltpu.VMEM((1024,), jnp.int32),
        pltpu.VMEM_SHARED((EP_TOT,), jnp.int32),
        pltpu.VMEM_SHARED((EP_TOT,), jnp.int32),
        pltpu.VMEM_SHARED((EP_TOT,), jnp.int32),
        pltpu.SemaphoreType.DMA,
    ],
)(_scb_body)


def _spmm_body(h0, h1, m0, m1, srcP, dstP, eidP, out0, out1,
               srcv, dstv, eidv, buf, mbuf, acc, semg, semm):
    c = lax.axis_index("c")
    s = lax.axis_index("s")

    def _zrow(r, carry):
        for q in range(8):
            buf[r, pl.ds(q * 16, 16)] = jnp.zeros((16,), jnp.float32)
        return carry

    lax.fori_loop(0, KE, _zrow, 0)
    rbase = s * RPT
    for j in range(4):
        pltpu.sync_copy(buf.at[pl.ds(0, 128)],
                        acc.at[pl.ds(rbase + j * 128, 128)])
    pltpu.sync_copy(buf.at[pl.ds(0, 112)], acc.at[pl.ds(rbase + 512, 112)])

    @pl.when(s == 0)
    def _():
        pltpu.sync_copy(buf.at[pl.ds(0, REM)], acc.at[pl.ds(REM_BASE, REM)])

    plsc.subcore_barrier()

    def _half(h, m, out):
        def _chunk(i, carry):
            eb = s * EPW + i * KE
            pltpu.sync_copy(srcP.at[pl.ds(eb, KE)], srcv)
            pltpu.sync_copy(dstP.at[pl.ds(eb, KE)], dstv)
            pltpu.sync_copy(eidP.at[pl.ds(eb, KE)], eidv)
            cm = pltpu.async_copy(m.at[eidv], mbuf, semm)
            cg = pltpu.async_copy(h.at[srcv], buf, semg)
            cm.wait()
            cg.wait()

            def _add(r, carry2):
                for q in range(8):
                    plsc.addupdate(buf.at[r, pl.ds(q * 16, 16)],
                                   mbuf[r, pl.ds(q * 16, 16)])
                return carry2

            lax.fori_loop(0, KE, _add, 0)
            pltpu.sync_copy(buf, acc.at[dstv], add=True)
            return carry

        lax.fori_loop(0, EPW // KE, _chunk, 0)
        plsc.subcore_barrier()
        pltpu.sync_copy(acc.at[pl.ds(rbase, RPT)], out.at[pl.ds(rbase, RPT)])

        @pl.when(s == 0)
        def _():
            pltpu.sync_copy(acc.at[pl.ds(REM_BASE, REMW)],
                            out.at[pl.ds(REM_BASE, REMW)])

    @pl.when(c == 0)
    def _():
        _half(h0, m0, out0)

    @pl.when(c == 1)
    def _():
        _half(h1, m1, out1)


_spmm = functools.partial(
    pl.kernel,
    out_type=[
        jax.ShapeDtypeStruct((N, HALF), jnp.float32),
        jax.ShapeDtypeStruct((N, HALF), jnp.float32),
    ],
    mesh=plsc.VectorSubcoreMesh(core_axis_name="c", subcore_axis_name="s"),
    scratch_types=[
        pltpu.VMEM((KE,), jnp.int32),
        pltpu.VMEM((KE,), jnp.int32),
        pltpu.VMEM((KE,), jnp.int32),
        pltpu.VMEM((KE, HALF), jnp.float32),
        pltpu.VMEM((KE, HALF), jnp.float32),
        pltpu.VMEM_SHARED((NACC, HALF), jnp.float32),
        pltpu.SemaphoreType.DMA,
        pltpu.SemaphoreType.DMA,
    ],
)(_spmm_body)


def kernel(x_r, e_r, x_p, e_p, edge_index_r, edge_index_p, seg_r, seg_p,
           r_dummy, p_dummy, W_in, b_in, W_e, W_h, b_h, Wq, Wk, W_out, b_out):
    xs = [x_r[0], x_r[1], x_p[0]]
    es = [e_r[0], e_r[1], e_p[0]]
    srcs = [edge_index_r[0, 0].astype(jnp.int32),
            edge_index_r[1, 0].astype(jnp.int32),
            edge_index_p[0, 0].astype(jnp.int32)]
    dsts = [edge_index_r[0, 1].astype(jnp.int32),
            edge_index_r[1, 1].astype(jnp.int32),
            edge_index_p[0, 1].astype(jnp.int32)]
    segs = [seg_r[0].astype(jnp.int32).reshape(N // BN, 1, BN),
            seg_r[1].astype(jnp.int32).reshape(N // BN, 1, BN),
            seg_p[0].astype(jnp.int32).reshape(N // BN, 1, BN)]

    b_in2 = b_in.reshape(1, EMB)

    buckets = []
    for g in range(3):
        pos3d, st2d, nc2d = _pos(dsts[g].reshape(E // BE, 1, BE))
        srcP, dstP, eidP = _scb(srcs[g], dsts[g], pos3d.reshape(E))
        buckets.append((srcP, dstP, eidP,
                        st2d.reshape(16), nc2d.reshape(16)))
    hs = [_embed(x, W_in, b_in2) for x in xs]
    ms = [[_mcalc(es[g], W_e[l]) for l in range(NL)] for g in range(3)]

    for l in range(NL):
        w_l = W_h[l]
        b_l = b_h[l].reshape(1, EMB)
        for g in range(3):
            srcP, dstP, eidP, st1, nc1 = buckets[g]
            h0 = hs[g][:, :HALF]
            h1 = hs[g][:, HALF:]
            a0, a1 = _spmm(h0, h1, ms[g][l][0], ms[g][l][1],
                           srcP, dstP, eidP)
            hs[g] = _dense(hs[g], a0, a1, w_l, b_l)

    means = [_readout(segs[g], hs[g]) for g in range(3)]
    return _att(means, Wq, Wk, W_out, b_out)
